# baseline (speedup 1.0000x reference)
import functools

import jax
import jax.numpy as jnp
from jax import lax
from jax.experimental import pallas as pl
from jax.experimental.pallas import tpu as pltpu
from jax.experimental.pallas import tpu_sc as plsc

B, L = 4, 8192
EMB_DIM = 1024

SC_WORKERS = 32
SC_ROWS_PER_W = L // SC_WORKERS
SC_CHUNK = 32
SC_NCHUNK = SC_ROWS_PER_W // SC_CHUNK


def _sc_bcast_body(table_hbm, out_hbm, buf, in_sem, out_sem):
    wid = lax.axis_index("s") * 2 + lax.axis_index("c")
    base = wid * SC_ROWS_PER_W

    def in_copy(c, s):
        return pltpu.make_async_copy(
            table_hbm.at[pl.ds(base + c * SC_CHUNK, SC_CHUNK), :],
            buf.at[s],
            in_sem.at[s],
        )

    def out_copy(c, s, b):
        return pltpu.make_async_copy(
            buf.at[s],
            out_hbm.at[pl.ds(b * L + base + c * SC_CHUNK, SC_CHUNK), :],
            out_sem.at[s, b],
        )

    in_copy(0, 0).start()
    for c in range(SC_NCHUNK):
        s = c % 2
        if c + 1 < SC_NCHUNK:
            if c >= 1:
                for b in range(B):
                    out_copy(c - 1, 1 - s, b).wait()
            in_copy(c + 1, 1 - s).start()
        in_copy(c, s).wait()
        for b in range(B):
            out_copy(c, s, b).start()
    for c in (SC_NCHUNK - 2, SC_NCHUNK - 1):
        for b in range(B):
            out_copy(c, c % 2, b).wait()


@functools.cache
def _sc_kernel():
    return pl.kernel(
        _sc_bcast_body,
        out_type=jax.ShapeDtypeStruct((B * L, EMB_DIM), jnp.float32),
        mesh=plsc.VectorSubcoreMesh(core_axis_name="c", subcore_axis_name="s"),
        scratch_types=[
            pltpu.VMEM((2, SC_CHUNK, EMB_DIM), jnp.float32),
            pltpu.SemaphoreType.DMA((2,)),
            pltpu.SemaphoreType.DMA((2, B)),
        ],
    )


def kernel(x, table):
    del x
    flat = _sc_kernel()(table)
    return flat.reshape(B, L, EMB_DIM)


# speedup vs baseline: 1.5194x; 1.5194x over previous
import jax
import jax.numpy as jnp
from jax.experimental import pallas as pl
from jax.experimental.pallas import tpu as pltpu

B, L = 4, 8192
EMB_DIM = 1024
CHUNK = 2048
NCHUNK = L // CHUNK
NSLOT = 4


def _dma_bcast_kernel(table_hbm, out_hbm, buf, in_sem, out_sem):
    i = pl.program_id(0)
    slot = jax.lax.rem(i, NSLOT)
    nslot = jax.lax.rem(i + 1, NSLOT)

    def in_copy(chunk, s):
        return pltpu.make_async_copy(
            table_hbm.at[pl.ds(chunk * CHUNK, CHUNK), :],
            buf.at[s],
            in_sem.at[s],
        )

    def out_copy(chunk, s, b):
        h = b // B
        bb = b % B
        return pltpu.make_async_copy(
            buf.at[s, pl.ds(h * (CHUNK // 2), CHUNK // 2), :],
            out_hbm.at[bb, pl.ds(chunk * CHUNK + h * (CHUNK // 2), CHUNK // 2), :],
            out_sem.at[s, b],
        )

    @pl.when(i == 0)
    def _():
        in_copy(0, 0).start()

    @pl.when(i + 1 < NCHUNK)
    def _():
        @pl.when(i + 1 >= NSLOT)
        def _():
            for b in range(2 * B):
                out_copy(i + 1 - NSLOT, nslot, b).wait()

        in_copy(i + 1, nslot).start()

    in_copy(i, slot).wait()
    for b in range(2 * B):
        out_copy(i, slot, b).start()

    @pl.when(i == NCHUNK - 1)
    def _():
        for j in range(min(NSLOT, NCHUNK)):
            chunk = NCHUNK - 1 - j
            for b in range(2 * B):
                out_copy(chunk, chunk % NSLOT, b).wait()


def kernel(x, table):
    del x
    return pl.pallas_call(
        _dma_bcast_kernel,
        grid=(NCHUNK,),
        in_specs=[pl.BlockSpec(memory_space=pltpu.MemorySpace.HBM)],
        out_specs=pl.BlockSpec(memory_space=pltpu.MemorySpace.HBM),
        out_shape=jax.ShapeDtypeStruct((B, L, EMB_DIM), table.dtype),
        scratch_shapes=[
            pltpu.VMEM((NSLOT, CHUNK, EMB_DIM), table.dtype),
            pltpu.SemaphoreType.DMA((NSLOT,)),
            pltpu.SemaphoreType.DMA((NSLOT, 2 * B)),
        ],
    )(table)


# speedup vs baseline: 1.5422x; 1.0150x over previous
import jax
import jax.numpy as jnp
from jax.experimental import pallas as pl
from jax.experimental.pallas import tpu as pltpu

B, L = 4, 8192
EMB_DIM = 1024
CHUNK = 4096
NCHUNK = L // CHUNK
NSLOT = 2


def _dma_bcast_kernel(table_hbm, out_hbm, buf, in_sem, out_sem):
    i = pl.program_id(0)
    slot = jax.lax.rem(i, NSLOT)
    nslot = jax.lax.rem(i + 1, NSLOT)

    def in_copy(chunk, s):
        return pltpu.make_async_copy(
            table_hbm.at[pl.ds(chunk * CHUNK, CHUNK), :],
            buf.at[s],
            in_sem.at[s],
        )

    def out_copy(chunk, s, b):
        h = b // B
        bb = b % B
        return pltpu.make_async_copy(
            buf.at[s, pl.ds(h * (CHUNK // 2), CHUNK // 2), :],
            out_hbm.at[bb, pl.ds(chunk * CHUNK + h * (CHUNK // 2), CHUNK // 2), :],
            out_sem.at[s, b],
        )

    @pl.when(i == 0)
    def _():
        in_copy(0, 0).start()

    @pl.when(i + 1 < NCHUNK)
    def _():
        @pl.when(i + 1 >= NSLOT)
        def _():
            for b in range(2 * B):
                out_copy(i + 1 - NSLOT, nslot, b).wait()

        in_copy(i + 1, nslot).start()

    in_copy(i, slot).wait()
    for b in range(2 * B):
        out_copy(i, slot, b).start()

    @pl.when(i == NCHUNK - 1)
    def _():
        for j in range(min(NSLOT, NCHUNK)):
            chunk = NCHUNK - 1 - j
            for b in range(2 * B):
                out_copy(chunk, chunk % NSLOT, b).wait()


def kernel(x, table):
    del x
    return pl.pallas_call(
        _dma_bcast_kernel,
        grid=(NCHUNK,),
        in_specs=[pl.BlockSpec(memory_space=pltpu.MemorySpace.HBM)],
        out_specs=pl.BlockSpec(memory_space=pltpu.MemorySpace.HBM),
        out_shape=jax.ShapeDtypeStruct((B, L, EMB_DIM), table.dtype),
        scratch_shapes=[
            pltpu.VMEM((NSLOT, CHUNK, EMB_DIM), table.dtype),
            pltpu.SemaphoreType.DMA((NSLOT,)),
            pltpu.SemaphoreType.DMA((NSLOT, 2 * B)),
        ],
    )(table)
